# baseline (speedup 1.0000x reference)
import functools

import jax
import jax.numpy as jnp
from jax import lax
from jax.experimental import pallas as pl
from jax.experimental.pallas import tpu as pltpu
from jax.experimental.pallas import tpu_sc as plsc

B, F, V, D = 4096, 22, 1000000, 16
H1, H2 = 400, 400
HP = 512
FD = F * D
IDX_CHUNK = 128


@functools.cache
def _sc_gather_kernel():
    info = plsc.get_sparse_core_info()
    nc, ns = info.num_cores, info.num_subcores
    nw = nc * ns
    rows_per_w = (B * F) // nw
    chunks = rows_per_w // IDX_CHUNK
    assert rows_per_w % IDX_CHUNK == 0

    mesh = plsc.VectorSubcoreMesh(core_axis_name="c", subcore_axis_name="s")

    @functools.partial(
        pl.kernel,
        mesh=mesh,
        out_type=(
            jax.ShapeDtypeStruct((B * F * D,), jnp.float32),
            jax.ShapeDtypeStruct((B * F,), jnp.float32),
        ),
        scratch_types=[
            pltpu.VMEM((rows_per_w,), jnp.int32),
            pltpu.VMEM((D * rows_per_w,), jnp.float32),
            pltpu.VMEM((rows_per_w * D,), jnp.float32),
            pltpu.VMEM((rows_per_w,), jnp.float32),
            pltpu.SemaphoreType.DMA,
            pltpu.SemaphoreType.DMA,
        ],
        compiler_params=pltpu.CompilerParams(use_tc_tiling_on_sc=False,
                                            needs_layout_passes=False),
    )
    def gather_kernel(ids_hbm, fmvt_hbm, fmw_hbm, emb_out, wg_out,
                      idx_v, gbuf, rows_v, w_v, sem_v, sem_w):
        wid = lax.axis_index("s") * nc + lax.axis_index("c")
        base = wid * rows_per_w
        pltpu.sync_copy(ids_hbm.at[pl.ds(base, rows_per_w)], idx_v)

        def chunk_body(c, carry):
            idx_c = idx_v.at[pl.ds(c * IDX_CHUNK, IDX_CHUNK)]
            copies = [pltpu.async_copy(fmw_hbm.at[idx_c],
                                       w_v.at[pl.ds(c * IDX_CHUNK, IDX_CHUNK)],
                                       sem_w)]
            for d in range(D):
                copies.append(pltpu.async_copy(
                    fmvt_hbm.at[d].at[idx_c],
                    gbuf.at[pl.ds(d * rows_per_w + c * IDX_CHUNK, IDX_CHUNK)],
                    sem_v))
            for cp in copies:
                cp.wait()
            return carry

        lax.fori_loop(0, chunks, chunk_body, 0, unroll=False)

        d_iota = lax.iota(jnp.int32, 16)
        ld_base = d_iota * rows_per_w
        st_base = d_iota

        def tr_body(j, jvec):
            row = plsc.load_gather(gbuf, [ld_base + jvec])
            plsc.store_scatter(rows_v, [jvec * D + st_base], row)
            return jvec + 1

        lax.fori_loop(0, rows_per_w, tr_body,
                      jnp.zeros((16,), jnp.int32), unroll=False)

        pltpu.sync_copy(rows_v, emb_out.at[pl.ds(base * D, rows_per_w * D)])
        pltpu.sync_copy(w_v, wg_out.at[pl.ds(base, rows_per_w)])

    return gather_kernel


BLK = 512


def _tc_body(emb_ref, vrep_ref, vals_ref, wg_ref, a_ref,
             w1_ref, b1_ref, w2_ref, b2_ref, w3_ref, scal_ref, out_ref):
    emb = emb_ref[...] * vrep_ref[...]
    s = jnp.dot(emb, a_ref[...], preferred_element_type=jnp.float32)
    y_wxx = 0.5 * (jnp.sum(s * s, axis=1, keepdims=True)
                   - jnp.sum(emb * emb, axis=1, keepdims=True))
    y_wx = jnp.sum(vals_ref[...] * wg_ref[...], axis=1, keepdims=True)
    h = jnp.dot(emb, w1_ref[...], preferred_element_type=jnp.float32)
    h = jnp.maximum(h + b1_ref[...], 0.0)
    h = jnp.dot(h, w2_ref[...], preferred_element_type=jnp.float32)
    h = jnp.maximum(h + b2_ref[...], 0.0)
    y_d = jnp.sum(h * w3_ref[...], axis=1, keepdims=True)
    y = y_wx + y_wxx + y_d + scal_ref[...]
    out_ref[...] = 1.0 / (1.0 + jnp.exp(-y))


def kernel(feat_ids, feat_vals, fm_b, fm_w, fm_v, W1, b1, W2, b2, W3, b3):
    ids_flat = feat_ids.astype(jnp.int32).reshape(B * F)
    emb_flat, wg_flat = _sc_gather_kernel()(ids_flat, fm_v.T, fm_w)
    emb = emb_flat.reshape(B, FD)
    wg = wg_flat.reshape(B, F)

    vrep = jnp.repeat(feat_vals, D, axis=1)
    a_mat = jnp.tile(jnp.eye(D, dtype=jnp.float32), (F, 1))
    w1p = jnp.pad(W1, ((0, 0), (0, HP - H1)))
    b1p = jnp.pad(b1, (0, HP - H1)).reshape(1, HP)
    w2p = jnp.pad(W2, ((0, HP - H1), (0, HP - H2)))
    b2p = jnp.pad(b2, (0, HP - H2)).reshape(1, HP)
    w3p = jnp.pad(W3[:, 0], (0, HP - H2)).reshape(1, HP)
    scal = (fm_b + b3).reshape(1, 1)

    full = lambda shape: pl.BlockSpec(shape, lambda i: (0, 0))
    preds = pl.pallas_call(
        _tc_body,
        grid=(B // BLK,),
        in_specs=[
            pl.BlockSpec((BLK, FD), lambda i: (i, 0)),
            pl.BlockSpec((BLK, FD), lambda i: (i, 0)),
            pl.BlockSpec((BLK, F), lambda i: (i, 0)),
            pl.BlockSpec((BLK, F), lambda i: (i, 0)),
            full((FD, D)),
            full((FD, HP)),
            full((1, HP)),
            full((HP, HP)),
            full((1, HP)),
            full((1, HP)),
            full((1, 1)),
        ],
        out_specs=pl.BlockSpec((BLK, 1), lambda i: (i, 0)),
        out_shape=jax.ShapeDtypeStruct((B, 1), jnp.float32),
        compiler_params=pltpu.CompilerParams(
            dimension_semantics=("parallel",)),
    )(emb, vrep, feat_vals, wg, a_mat, w1p, b1p, w2p, b2p, w3p, scal)
    return preds.reshape(-1)


# speedup vs baseline: 5.8317x; 5.8317x over previous
import functools

import jax
import jax.numpy as jnp
from jax import lax
from jax.experimental import pallas as pl
from jax.experimental.pallas import tpu as pltpu
from jax.experimental.pallas import tpu_sc as plsc

B, F, V, D = 4096, 22, 1000000, 16
H1, H2 = 400, 400
HP = 512
FD = F * D
IDX_CHUNK = 128


VP = 1007616
DET_COLS = VP // 8


def _detile_body(in_ref, *out_refs):
    g = pl.program_id(1)
    for dd in range(D):
        @pl.when(g == dd // 8)
        def _(dd=dd):
            out_refs[dd][...] = in_ref[dd % 8, :]


def _tc_detile(fmvt_p):
    return pl.pallas_call(
        _detile_body,
        grid=(8, 2),
        in_specs=[pl.BlockSpec((8, DET_COLS), lambda c, g: (g, c))],
        out_specs=[pl.BlockSpec((DET_COLS,), lambda c, g: (c,))
                   for _ in range(D)],
        out_shape=[jax.ShapeDtypeStruct((VP,), jnp.float32)
                   for _ in range(D)],
        compiler_params=pltpu.CompilerParams(
            dimension_semantics=("arbitrary", "arbitrary")),
    )(fmvt_p)


@functools.cache
def _sc_gather_kernel():
    info = plsc.get_sparse_core_info()
    nc, ns = info.num_cores, info.num_subcores
    nw = nc * ns
    rows_per_w = (B * F) // nw
    chunks = rows_per_w // IDX_CHUNK
    assert rows_per_w % IDX_CHUNK == 0

    mesh = plsc.VectorSubcoreMesh(core_axis_name="c", subcore_axis_name="s")

    @functools.partial(
        pl.kernel,
        mesh=mesh,
        out_type=(
            jax.ShapeDtypeStruct((B * F * D,), jnp.float32),
            jax.ShapeDtypeStruct((B * F,), jnp.float32),
        ),
        scratch_types=[
            pltpu.VMEM((rows_per_w,), jnp.int32),
            pltpu.VMEM((D * rows_per_w,), jnp.float32),
            pltpu.VMEM((rows_per_w * D,), jnp.float32),
            pltpu.VMEM((rows_per_w,), jnp.float32),
            pltpu.SemaphoreType.DMA,
            pltpu.SemaphoreType.DMA,
        ],
        compiler_params=pltpu.CompilerParams(use_tc_tiling_on_sc=False,
                                            needs_layout_passes=False),
    )
    def gather_kernel(ids_hbm, *rest):
        tabs = rest[:D]
        (fmw_hbm, emb_out, wg_out,
         idx_v, gbuf, rows_v, w_v, sem_v, sem_w) = rest[D:]
        wid = lax.axis_index("s") * nc + lax.axis_index("c")
        base = wid * rows_per_w
        pltpu.sync_copy(ids_hbm.at[pl.ds(base, rows_per_w)], idx_v)

        def chunk_body(c, carry):
            idx_c = idx_v.at[pl.ds(c * IDX_CHUNK, IDX_CHUNK)]
            copies = [pltpu.async_copy(fmw_hbm.at[idx_c],
                                       w_v.at[pl.ds(c * IDX_CHUNK, IDX_CHUNK)],
                                       sem_w)]
            for d in range(D):
                copies.append(pltpu.async_copy(
                    tabs[d].at[idx_c],
                    gbuf.at[pl.ds(d * rows_per_w + c * IDX_CHUNK, IDX_CHUNK)],
                    sem_v))
            for cp in copies:
                cp.wait()
            return carry

        lax.fori_loop(0, chunks, chunk_body, 0, unroll=False)

        d_iota = lax.iota(jnp.int32, 16)
        ld_base = d_iota * rows_per_w
        st_base = d_iota

        def tr_body(j, jvec):
            row = plsc.load_gather(gbuf, [ld_base + jvec])
            plsc.store_scatter(rows_v, [jvec * D + st_base], row)
            return jvec + 1

        lax.fori_loop(0, rows_per_w, tr_body,
                      jnp.zeros((16,), jnp.int32), unroll=False)

        pltpu.sync_copy(rows_v, emb_out.at[pl.ds(base * D, rows_per_w * D)])
        pltpu.sync_copy(w_v, wg_out.at[pl.ds(base, rows_per_w)])

    return gather_kernel


BLK = 512


def _tc_body(emb_ref, vrep_ref, vals_ref, wg_ref, a_ref,
             w1_ref, b1_ref, w2_ref, b2_ref, w3_ref, scal_ref, out_ref):
    emb = emb_ref[...] * vrep_ref[...]
    s = jnp.dot(emb, a_ref[...], preferred_element_type=jnp.float32)
    y_wxx = 0.5 * (jnp.sum(s * s, axis=1, keepdims=True)
                   - jnp.sum(emb * emb, axis=1, keepdims=True))
    y_wx = jnp.sum(vals_ref[...] * wg_ref[...], axis=1, keepdims=True)
    h = jnp.dot(emb, w1_ref[...], preferred_element_type=jnp.float32)
    h = jnp.maximum(h + b1_ref[...], 0.0)
    h = jnp.dot(h, w2_ref[...], preferred_element_type=jnp.float32)
    h = jnp.maximum(h + b2_ref[...], 0.0)
    y_d = jnp.sum(h * w3_ref[...], axis=1, keepdims=True)
    y = y_wx + y_wxx + y_d + scal_ref[...]
    out_ref[...] = 1.0 / (1.0 + jnp.exp(-y))


def kernel(feat_ids, feat_vals, fm_b, fm_w, fm_v, W1, b1, W2, b2, W3, b3):
    ids_flat = feat_ids.astype(jnp.int32).reshape(B * F)
    fmvt_p = jnp.pad(fm_v.T, ((0, 0), (0, VP - V)))
    tabs = _tc_detile(fmvt_p)
    emb_flat, wg_flat = _sc_gather_kernel()(ids_flat, *tabs, fm_w)
    emb = emb_flat.reshape(B, FD)
    wg = wg_flat.reshape(B, F)

    vrep = jnp.repeat(feat_vals, D, axis=1)
    a_mat = jnp.tile(jnp.eye(D, dtype=jnp.float32), (F, 1))
    w1p = jnp.pad(W1, ((0, 0), (0, HP - H1)))
    b1p = jnp.pad(b1, (0, HP - H1)).reshape(1, HP)
    w2p = jnp.pad(W2, ((0, HP - H1), (0, HP - H2)))
    b2p = jnp.pad(b2, (0, HP - H2)).reshape(1, HP)
    w3p = jnp.pad(W3[:, 0], (0, HP - H2)).reshape(1, HP)
    scal = (fm_b + b3).reshape(1, 1)

    full = lambda shape: pl.BlockSpec(shape, lambda i: (0, 0))
    preds = pl.pallas_call(
        _tc_body,
        grid=(B // BLK,),
        in_specs=[
            pl.BlockSpec((BLK, FD), lambda i: (i, 0)),
            pl.BlockSpec((BLK, FD), lambda i: (i, 0)),
            pl.BlockSpec((BLK, F), lambda i: (i, 0)),
            pl.BlockSpec((BLK, F), lambda i: (i, 0)),
            full((FD, D)),
            full((FD, HP)),
            full((1, HP)),
            full((HP, HP)),
            full((1, HP)),
            full((1, HP)),
            full((1, 1)),
        ],
        out_specs=pl.BlockSpec((BLK, 1), lambda i: (i, 0)),
        out_shape=jax.ShapeDtypeStruct((B, 1), jnp.float32),
        compiler_params=pltpu.CompilerParams(
            dimension_semantics=("parallel",)),
    )(emb, vrep, feat_vals, wg, a_mat, w1p, b1p, w2p, b2p, w3p, scal)
    return preds.reshape(-1)


# speedup vs baseline: 6.9948x; 1.1994x over previous
import functools

import jax
import jax.numpy as jnp
from jax import lax
from jax.experimental import pallas as pl
from jax.experimental.pallas import tpu as pltpu
from jax.experimental.pallas import tpu_sc as plsc

B, F, V, D = 4096, 22, 1000000, 16
H1, H2 = 400, 400
HP = 512
FD = F * D
IDX_CHUNK = 128


VP = 1007616
DET_COLS = VP // 8


def _detile_body(in_ref, *out_refs):
    g = pl.program_id(1)
    for dd in range(D):
        @pl.when(g == dd // 8)
        def _(dd=dd):
            out_refs[dd][...] = in_ref[dd % 8, :]


def _tc_detile(fmvt):
    return pl.pallas_call(
        _detile_body,
        grid=(8, 2),
        in_specs=[pl.BlockSpec((8, DET_COLS), lambda c, g: (g, c))],
        out_specs=[pl.BlockSpec((DET_COLS,), lambda c, g: (c,))
                   for _ in range(D)],
        out_shape=[jax.ShapeDtypeStruct((VP,), jnp.float32)
                   for _ in range(D)],
        compiler_params=pltpu.CompilerParams(
            dimension_semantics=("arbitrary", "arbitrary")),
    )(fmvt)


@functools.cache
def _sc_gather_kernel():
    info = plsc.get_sparse_core_info()
    nc, ns = info.num_cores, info.num_subcores
    nw = nc * ns
    rows_per_w = (B * F) // nw
    chunks = rows_per_w // IDX_CHUNK
    assert rows_per_w % IDX_CHUNK == 0

    mesh = plsc.VectorSubcoreMesh(core_axis_name="c", subcore_axis_name="s")

    @functools.partial(
        pl.kernel,
        mesh=mesh,
        out_type=(
            jax.ShapeDtypeStruct((B * F * D,), jnp.float32),
            jax.ShapeDtypeStruct((B * F,), jnp.float32),
        ),
        scratch_types=[
            pltpu.VMEM((rows_per_w,), jnp.int32),
            pltpu.VMEM((D * rows_per_w,), jnp.float32),
            pltpu.VMEM((rows_per_w * D,), jnp.float32),
            pltpu.VMEM((rows_per_w,), jnp.float32),
            pltpu.SemaphoreType.DMA,
            pltpu.SemaphoreType.DMA,
        ],
        compiler_params=pltpu.CompilerParams(use_tc_tiling_on_sc=False,
                                            needs_layout_passes=False),
    )
    def gather_kernel(ids_hbm, *rest):
        tabs = rest[:D]
        (fmw_hbm, emb_out, wg_out,
         idx_v, gbuf, rows_v, w_v, sem_v, sem_w) = rest[D:]
        wid = lax.axis_index("s") * nc + lax.axis_index("c")
        base = wid * rows_per_w
        pltpu.sync_copy(ids_hbm.at[pl.ds(base, rows_per_w)], idx_v)

        def chunk_body(c, carry):
            idx_c = idx_v.at[pl.ds(c * IDX_CHUNK, IDX_CHUNK)]
            copies = [pltpu.async_copy(fmw_hbm.at[idx_c],
                                       w_v.at[pl.ds(c * IDX_CHUNK, IDX_CHUNK)],
                                       sem_w)]
            for d in range(D):
                copies.append(pltpu.async_copy(
                    tabs[d].at[idx_c],
                    gbuf.at[pl.ds(d * rows_per_w + c * IDX_CHUNK, IDX_CHUNK)],
                    sem_v))
            for cp in copies:
                cp.wait()
            return carry

        lax.fori_loop(0, chunks, chunk_body, 0, unroll=False)

        d_iota = lax.iota(jnp.int32, 16)
        ld_base = d_iota * rows_per_w
        st_base = d_iota

        def tr_body(j, jvec):
            row = plsc.load_gather(gbuf, [ld_base + jvec])
            plsc.store_scatter(rows_v, [jvec * D + st_base], row)
            return jvec + 1

        lax.fori_loop(0, rows_per_w, tr_body,
                      jnp.zeros((16,), jnp.int32), unroll=False)

        pltpu.sync_copy(rows_v, emb_out.at[pl.ds(base * D, rows_per_w * D)])
        pltpu.sync_copy(w_v, wg_out.at[pl.ds(base, rows_per_w)])

    return gather_kernel


BLK = 512


def _tc_body(emb_ref, vrep_ref, vals_ref, wg_ref, a_ref,
             w1_ref, b1_ref, w2_ref, b2_ref, w3_ref, scal_ref, out_ref):
    emb = emb_ref[...] * vrep_ref[...]
    s = jnp.dot(emb, a_ref[...], preferred_element_type=jnp.float32)
    y_wxx = 0.5 * (jnp.sum(s * s, axis=1, keepdims=True)
                   - jnp.sum(emb * emb, axis=1, keepdims=True))
    y_wx = jnp.sum(vals_ref[...] * wg_ref[...], axis=1, keepdims=True)
    h = jnp.dot(emb, w1_ref[...], preferred_element_type=jnp.float32)
    h = jnp.maximum(h + b1_ref[...], 0.0)
    h = jnp.dot(h, w2_ref[...], preferred_element_type=jnp.float32)
    h = jnp.maximum(h + b2_ref[...], 0.0)
    y_d = jnp.sum(h * w3_ref[...], axis=1, keepdims=True)
    y = y_wx + y_wxx + y_d + scal_ref[...]
    out_ref[...] = 1.0 / (1.0 + jnp.exp(-y))


def kernel(feat_ids, feat_vals, fm_b, fm_w, fm_v, W1, b1, W2, b2, W3, b3):
    ids_flat = feat_ids.astype(jnp.int32).reshape(B * F)
    tabs = _tc_detile(fm_v.T)
    emb_flat, wg_flat = _sc_gather_kernel()(ids_flat, *tabs, fm_w)
    emb = emb_flat.reshape(B, FD)
    wg = wg_flat.reshape(B, F)

    vrep = jnp.repeat(feat_vals, D, axis=1)
    a_mat = jnp.tile(jnp.eye(D, dtype=jnp.float32), (F, 1))
    w1p = jnp.pad(W1, ((0, 0), (0, HP - H1)))
    b1p = jnp.pad(b1, (0, HP - H1)).reshape(1, HP)
    w2p = jnp.pad(W2, ((0, HP - H1), (0, HP - H2)))
    b2p = jnp.pad(b2, (0, HP - H2)).reshape(1, HP)
    w3p = jnp.pad(W3[:, 0], (0, HP - H2)).reshape(1, HP)
    scal = (fm_b + b3).reshape(1, 1)

    full = lambda shape: pl.BlockSpec(shape, lambda i: (0, 0))
    preds = pl.pallas_call(
        _tc_body,
        grid=(B // BLK,),
        in_specs=[
            pl.BlockSpec((BLK, FD), lambda i: (i, 0)),
            pl.BlockSpec((BLK, FD), lambda i: (i, 0)),
            pl.BlockSpec((BLK, F), lambda i: (i, 0)),
            pl.BlockSpec((BLK, F), lambda i: (i, 0)),
            full((FD, D)),
            full((FD, HP)),
            full((1, HP)),
            full((HP, HP)),
            full((1, HP)),
            full((1, HP)),
            full((1, 1)),
        ],
        out_specs=pl.BlockSpec((BLK, 1), lambda i: (i, 0)),
        out_shape=jax.ShapeDtypeStruct((B, 1), jnp.float32),
        compiler_params=pltpu.CompilerParams(
            dimension_semantics=("parallel",)),
    )(emb, vrep, feat_vals, wg, a_mat, w1p, b1p, w2p, b2p, w3p, scal)
    return preds.reshape(-1)


# speedup vs baseline: 7.0535x; 1.0084x over previous
import functools

import jax
import jax.numpy as jnp
from jax import lax
from jax.experimental import pallas as pl
from jax.experimental.pallas import tpu as pltpu
from jax.experimental.pallas import tpu_sc as plsc

B, F, V, D = 4096, 22, 1000000, 16
H1, H2 = 400, 400
HP = 512
FD = F * D
IDX_CHUNK = 128


VP = 1007616
DET_COLS = VP // 8


def _detile_body(in_ref, *out_refs):
    g = pl.program_id(1)
    for dd in range(D):
        @pl.when(g == dd // 8)
        def _(dd=dd):
            out_refs[dd][...] = in_ref[dd % 8, :]


def _tc_detile(fmvt):
    return pl.pallas_call(
        _detile_body,
        grid=(8, 2),
        in_specs=[pl.BlockSpec((8, DET_COLS), lambda c, g: (g, c))],
        out_specs=[pl.BlockSpec((DET_COLS,), lambda c, g: (c,))
                   for _ in range(D)],
        out_shape=[jax.ShapeDtypeStruct((VP,), jnp.float32)
                   for _ in range(D)],
        compiler_params=pltpu.CompilerParams(
            dimension_semantics=("arbitrary", "arbitrary")),
    )(fmvt)


@functools.cache
def _sc_gather_kernel():
    info = plsc.get_sparse_core_info()
    nc, ns = info.num_cores, info.num_subcores
    nw = nc * ns
    rows_per_w = (B * F) // nw
    chunks = rows_per_w // IDX_CHUNK
    assert rows_per_w % IDX_CHUNK == 0

    mesh = plsc.VectorSubcoreMesh(core_axis_name="c", subcore_axis_name="s")

    @functools.partial(
        pl.kernel,
        mesh=mesh,
        out_type=(
            jax.ShapeDtypeStruct((B * F * D,), jnp.float32),
            jax.ShapeDtypeStruct((B * F,), jnp.float32),
        ),
        scratch_types=[
            pltpu.VMEM((rows_per_w,), jnp.int32),
            pltpu.VMEM((D * rows_per_w,), jnp.float32),
            pltpu.VMEM((rows_per_w * D,), jnp.float32),
            pltpu.VMEM((rows_per_w,), jnp.float32),
            pltpu.SemaphoreType.DMA,
            pltpu.SemaphoreType.DMA,
        ],
        compiler_params=pltpu.CompilerParams(use_tc_tiling_on_sc=False,
                                            needs_layout_passes=False),
    )
    def gather_kernel(ids_hbm, *rest):
        tabs = rest[:D]
        (fmw_hbm, emb_out, wg_out,
         idx_v, gbuf, rows_v, w_v, sem_v, sem_w) = rest[D:]
        wid = lax.axis_index("s") * nc + lax.axis_index("c")
        base = wid * rows_per_w
        pltpu.sync_copy(ids_hbm.at[pl.ds(base, rows_per_w)], idx_v)

        def issue(c):
            idx_c = idx_v.at[pl.ds(c * IDX_CHUNK, IDX_CHUNK)]
            pltpu.async_copy(fmw_hbm.at[idx_c],
                             w_v.at[pl.ds(c * IDX_CHUNK, IDX_CHUNK)], sem_w)
            for d in range(D):
                pltpu.async_copy(
                    tabs[d].at[idx_c],
                    gbuf.at[pl.ds(d * rows_per_w + c * IDX_CHUNK, IDX_CHUNK)],
                    sem_v)

        def drain(c):
            pltpu.make_async_copy(
                fmw_hbm.at[pl.ds(0, IDX_CHUNK)],
                w_v.at[pl.ds(c * IDX_CHUNK, IDX_CHUNK)], sem_w).wait()
            for d in range(D):
                pltpu.make_async_copy(
                    tabs[d].at[pl.ds(0, IDX_CHUNK)],
                    gbuf.at[pl.ds(d * rows_per_w + c * IDX_CHUNK, IDX_CHUNK)],
                    sem_v).wait()

        def chunk_body(c, carry):
            issue(c)

            @pl.when(c > 0)
            def _():
                drain(c - 1)
            return carry

        lax.fori_loop(0, chunks, chunk_body, 0, unroll=False)
        drain(chunks - 1)

        d_iota = lax.iota(jnp.int32, 16)
        ld_base = d_iota * rows_per_w
        st_base = d_iota

        def tr_body(j, jvec):
            row = plsc.load_gather(gbuf, [ld_base + jvec])
            plsc.store_scatter(rows_v, [jvec * D + st_base], row)
            return jvec + 1

        lax.fori_loop(0, rows_per_w, tr_body,
                      jnp.zeros((16,), jnp.int32), unroll=4)

        pltpu.sync_copy(rows_v, emb_out.at[pl.ds(base * D, rows_per_w * D)])
        pltpu.sync_copy(w_v, wg_out.at[pl.ds(base, rows_per_w)])

    return gather_kernel


BLK = 512


def _tc_body(emb_ref, vrep_ref, vals_ref, wg_ref, a_ref,
             w1_ref, b1_ref, w2_ref, b2_ref, w3_ref, scal_ref, out_ref):
    emb = emb_ref[...] * vrep_ref[...]
    s = jnp.dot(emb, a_ref[...], preferred_element_type=jnp.float32)
    y_wxx = 0.5 * (jnp.sum(s * s, axis=1, keepdims=True)
                   - jnp.sum(emb * emb, axis=1, keepdims=True))
    y_wx = jnp.sum(vals_ref[...] * wg_ref[...], axis=1, keepdims=True)
    h = jnp.dot(emb, w1_ref[...], preferred_element_type=jnp.float32)
    h = jnp.maximum(h + b1_ref[...], 0.0)
    h = jnp.dot(h, w2_ref[...], preferred_element_type=jnp.float32)
    h = jnp.maximum(h + b2_ref[...], 0.0)
    y_d = jnp.sum(h * w3_ref[...], axis=1, keepdims=True)
    y = y_wx + y_wxx + y_d + scal_ref[...]
    out_ref[...] = 1.0 / (1.0 + jnp.exp(-y))


def kernel(feat_ids, feat_vals, fm_b, fm_w, fm_v, W1, b1, W2, b2, W3, b3):
    ids_flat = feat_ids.astype(jnp.int32).reshape(B * F)
    tabs = _tc_detile(fm_v.T)
    emb_flat, wg_flat = _sc_gather_kernel()(ids_flat, *tabs, fm_w)
    emb = emb_flat.reshape(B, FD)
    wg = wg_flat.reshape(B, F)

    vrep = jnp.repeat(feat_vals, D, axis=1)
    a_mat = jnp.tile(jnp.eye(D, dtype=jnp.float32), (F, 1))
    w1p = jnp.pad(W1, ((0, 0), (0, HP - H1)))
    b1p = jnp.pad(b1, (0, HP - H1)).reshape(1, HP)
    w2p = jnp.pad(W2, ((0, HP - H1), (0, HP - H2)))
    b2p = jnp.pad(b2, (0, HP - H2)).reshape(1, HP)
    w3p = jnp.pad(W3[:, 0], (0, HP - H2)).reshape(1, HP)
    scal = (fm_b + b3).reshape(1, 1)

    full = lambda shape: pl.BlockSpec(shape, lambda i: (0, 0))
    preds = pl.pallas_call(
        _tc_body,
        grid=(B // BLK,),
        in_specs=[
            pl.BlockSpec((BLK, FD), lambda i: (i, 0)),
            pl.BlockSpec((BLK, FD), lambda i: (i, 0)),
            pl.BlockSpec((BLK, F), lambda i: (i, 0)),
            pl.BlockSpec((BLK, F), lambda i: (i, 0)),
            full((FD, D)),
            full((FD, HP)),
            full((1, HP)),
            full((HP, HP)),
            full((1, HP)),
            full((1, HP)),
            full((1, 1)),
        ],
        out_specs=pl.BlockSpec((BLK, 1), lambda i: (i, 0)),
        out_shape=jax.ShapeDtypeStruct((B, 1), jnp.float32),
        compiler_params=pltpu.CompilerParams(
            dimension_semantics=("parallel",)),
    )(emb, vrep, feat_vals, wg, a_mat, w1p, b1p, w2p, b2p, w3p, scal)
    return preds.reshape(-1)


# speedup vs baseline: 7.5347x; 1.0682x over previous
import functools

import numpy as np

import jax
import jax.numpy as jnp
from jax import lax
from jax.experimental import pallas as pl
from jax.experimental.pallas import tpu as pltpu
from jax.experimental.pallas import tpu_sc as plsc

B, F, V, D = 4096, 22, 1000000, 16
H1, H2 = 400, 400
HP = 512
FD = F * D
DH = D // 2
FDH = F * DH
IDX_CHUNK = 128

VP = 1007616
DET_COLS = VP // 8


def _detile_body(in_ref, *out_refs):
    for dd in range(DH):
        out_refs[dd][...] = in_ref[dd, :]


def _tc_detile(fmvt, half):
    return pl.pallas_call(
        _detile_body,
        grid=(8,),
        in_specs=[pl.BlockSpec((DH, DET_COLS), lambda c, h=half: (h, c))],
        out_specs=[pl.BlockSpec((DET_COLS,), lambda c: (c,))
                   for _ in range(DH)],
        out_shape=[jax.ShapeDtypeStruct((VP,), jnp.float32)
                   for _ in range(DH)],
        compiler_params=pltpu.CompilerParams(
            dimension_semantics=("arbitrary",)),
    )(fmvt)


@functools.cache
def _sc_gather_kernel(with_w):
    info = plsc.get_sparse_core_info()
    nc, ns = info.num_cores, info.num_subcores
    nw = nc * ns
    rows_per_w = (B * F) // nw
    chunks = rows_per_w // IDX_CHUNK
    assert rows_per_w % IDX_CHUNK == 0

    mesh = plsc.VectorSubcoreMesh(core_axis_name="c", subcore_axis_name="s")

    out_type = [jax.ShapeDtypeStruct((B * F * DH,), jnp.float32)]
    scratch = [
        pltpu.VMEM((rows_per_w,), jnp.int32),
        pltpu.VMEM((DH * rows_per_w,), jnp.float32),
        pltpu.VMEM((rows_per_w * DH,), jnp.float32),
        pltpu.SemaphoreType.DMA,
    ]
    if with_w:
        out_type.append(jax.ShapeDtypeStruct((B * F,), jnp.float32))
        scratch += [pltpu.VMEM((rows_per_w,), jnp.float32),
                    pltpu.SemaphoreType.DMA]

    @functools.partial(
        pl.kernel,
        mesh=mesh,
        out_type=tuple(out_type),
        scratch_types=scratch,
        compiler_params=pltpu.CompilerParams(use_tc_tiling_on_sc=False,
                                            needs_layout_passes=False),
    )
    def gather_kernel(ids_hbm, *rest):
        tabs = rest[:DH]
        if with_w:
            (fmw_hbm, emb_out, wg_out,
             idx_v, gbuf, rows_v, sem_v, w_v, sem_w) = rest[DH:]
        else:
            (emb_out, idx_v, gbuf, rows_v, sem_v) = rest[DH:]
        wid = lax.axis_index("s") * nc + lax.axis_index("c")
        base = wid * rows_per_w
        pltpu.sync_copy(ids_hbm.at[pl.ds(base, rows_per_w)], idx_v)

        def issue(c):
            idx_c = idx_v.at[pl.ds(c * IDX_CHUNK, IDX_CHUNK)]
            if with_w:
                pltpu.async_copy(fmw_hbm.at[idx_c],
                                 w_v.at[pl.ds(c * IDX_CHUNK, IDX_CHUNK)],
                                 sem_w)
            for d in range(DH):
                pltpu.async_copy(
                    tabs[d].at[idx_c],
                    gbuf.at[pl.ds(d * rows_per_w + c * IDX_CHUNK, IDX_CHUNK)],
                    sem_v)

        def drain(c):
            if with_w:
                pltpu.make_async_copy(
                    fmw_hbm.at[pl.ds(0, IDX_CHUNK)],
                    w_v.at[pl.ds(c * IDX_CHUNK, IDX_CHUNK)], sem_w).wait()
            for d in range(DH):
                pltpu.make_async_copy(
                    tabs[d].at[pl.ds(0, IDX_CHUNK)],
                    gbuf.at[pl.ds(d * rows_per_w + c * IDX_CHUNK, IDX_CHUNK)],
                    sem_v).wait()

        def chunk_body(c, carry):
            issue(c)

            @pl.when(c > 0)
            def _():
                drain(c - 1)
            return carry

        lax.fori_loop(0, chunks, chunk_body, 0, unroll=False)
        drain(chunks - 1)

        d_iota = lax.iota(jnp.int32, 16)
        dmod = d_iota % DH
        jsel = (d_iota >= DH).astype(jnp.int32)
        ld_base = dmod * rows_per_w

        def tr_body(j, jvec):
            j2 = jvec + jsel
            row = plsc.load_gather(gbuf, [ld_base + j2])
            plsc.store_scatter(rows_v, [j2 * DH + dmod], row)
            return jvec + 2

        lax.fori_loop(0, rows_per_w // 2, tr_body,
                      jnp.zeros((16,), jnp.int32), unroll=4)

        pltpu.sync_copy(rows_v, emb_out.at[pl.ds(base * DH, rows_per_w * DH)])
        if with_w:
            pltpu.sync_copy(w_v, wg_out.at[pl.ds(base, rows_per_w)])

    return gather_kernel


BLK = 512


def _tc_body(lo_ref, hi_ref, vrep_ref, vals_ref, wg_ref, a_ref,
             w1_ref, b1_ref, w2_ref, b2_ref, w3_ref, scal_ref, out_ref):
    emb = jnp.concatenate([lo_ref[...], hi_ref[...]], axis=1) * vrep_ref[...]
    s = jnp.dot(emb, a_ref[...], preferred_element_type=jnp.float32)
    y_wxx = 0.5 * (jnp.sum(s * s, axis=1, keepdims=True)
                   - jnp.sum(emb * emb, axis=1, keepdims=True))
    y_wx = jnp.sum(vals_ref[...] * wg_ref[...], axis=1, keepdims=True)
    h = jnp.dot(emb, w1_ref[...], preferred_element_type=jnp.float32)
    h = jnp.maximum(h + b1_ref[...], 0.0)
    h = jnp.dot(h, w2_ref[...], preferred_element_type=jnp.float32)
    h = jnp.maximum(h + b2_ref[...], 0.0)
    y_d = jnp.sum(h * w3_ref[...], axis=1, keepdims=True)
    y = y_wx + y_wxx + y_d + scal_ref[...]
    out_ref[...] = 1.0 / (1.0 + jnp.exp(-y))


_PERM = np.array([f * D + d for f in range(F) for d in range(DH)]
                 + [f * D + d for f in range(F) for d in range(DH, D)],
                 dtype=np.int32)


def kernel(feat_ids, feat_vals, fm_b, fm_w, fm_v, W1, b1, W2, b2, W3, b3):
    ids_flat = feat_ids.astype(jnp.int32).reshape(B * F)
    fmvt = fm_v.T

    tabs_lo = _tc_detile(fmvt, 0)
    emb_lo, wg_flat = _sc_gather_kernel(True)(ids_flat, *tabs_lo, fm_w)
    tabs_hi = _tc_detile(fmvt, 1)
    (emb_hi,) = _sc_gather_kernel(False)(ids_flat, *tabs_hi)

    lo = emb_lo.reshape(B, FDH)
    hi = emb_hi.reshape(B, FDH)
    wg = wg_flat.reshape(B, F)

    vrep = jnp.tile(jnp.repeat(feat_vals, DH, axis=1), (1, 2))
    a_mat = jnp.tile(jnp.eye(D, dtype=jnp.float32), (F, 1))[_PERM]
    w1p = jnp.pad(W1, ((0, 0), (0, HP - H1)))[_PERM]
    b1p = jnp.pad(b1, (0, HP - H1)).reshape(1, HP)
    w2p = jnp.pad(W2, ((0, HP - H1), (0, HP - H2)))
    b2p = jnp.pad(b2, (0, HP - H2)).reshape(1, HP)
    w3p = jnp.pad(W3[:, 0], (0, HP - H2)).reshape(1, HP)
    scal = (fm_b + b3).reshape(1, 1)

    full = lambda shape: pl.BlockSpec(shape, lambda i: (0, 0))
    preds = pl.pallas_call(
        _tc_body,
        grid=(B // BLK,),
        in_specs=[
            pl.BlockSpec((BLK, FDH), lambda i: (i, 0)),
            pl.BlockSpec((BLK, FDH), lambda i: (i, 0)),
            pl.BlockSpec((BLK, FD), lambda i: (i, 0)),
            pl.BlockSpec((BLK, F), lambda i: (i, 0)),
            pl.BlockSpec((BLK, F), lambda i: (i, 0)),
            full((FD, D)),
            full((FD, HP)),
            full((1, HP)),
            full((HP, HP)),
            full((1, HP)),
            full((1, HP)),
            full((1, 1)),
        ],
        out_specs=pl.BlockSpec((BLK, 1), lambda i: (i, 0)),
        out_shape=jax.ShapeDtypeStruct((B, 1), jnp.float32),
        compiler_params=pltpu.CompilerParams(
            dimension_semantics=("parallel",)),
    )(lo, hi, vrep, feat_vals, wg, a_mat, w1p, b1p, w2p, b2p, w3p, scal)
    return preds.reshape(-1)


# speedup vs baseline: 7.6202x; 1.0113x over previous
import functools

import numpy as np

import jax
import jax.numpy as jnp
from jax import lax
from jax.experimental import pallas as pl
from jax.experimental.pallas import tpu as pltpu
from jax.experimental.pallas import tpu_sc as plsc

B, F, V, D = 4096, 22, 1000000, 16
H1, H2 = 400, 400
HP = 512
FD = F * D
DH = D // 2
FDH = F * DH
IDX_CHUNK = 128

VP = 1007616
DET_COLS = VP // 8


def _detile_body(in_ref, *out_refs):
    for dd in range(DH):
        out_refs[dd][...] = in_ref[dd, :]


def _tc_detile(fmvt, half):
    return pl.pallas_call(
        _detile_body,
        grid=(8,),
        in_specs=[pl.BlockSpec((DH, DET_COLS), lambda c, h=half: (h, c))],
        out_specs=[pl.BlockSpec((DET_COLS,), lambda c: (c,))
                   for _ in range(DH)],
        out_shape=[jax.ShapeDtypeStruct((VP,), jnp.float32)
                   for _ in range(DH)],
        compiler_params=pltpu.CompilerParams(
            dimension_semantics=("arbitrary",)),
    )(fmvt)


@functools.cache
def _sc_gather_kernel(with_w):
    info = plsc.get_sparse_core_info()
    nc, ns = info.num_cores, info.num_subcores
    nw = nc * ns
    rows_per_w = (B * F) // nw
    chunks = rows_per_w // IDX_CHUNK
    assert rows_per_w % IDX_CHUNK == 0

    mesh = plsc.VectorSubcoreMesh(core_axis_name="c", subcore_axis_name="s")

    out_type = [jax.ShapeDtypeStruct((B * F * DH,), jnp.float32)]
    scratch = [
        pltpu.VMEM((rows_per_w,), jnp.int32),
        pltpu.VMEM((DH * rows_per_w,), jnp.float32),
        pltpu.VMEM((rows_per_w * DH,), jnp.float32),
        pltpu.SemaphoreType.DMA,
    ]
    if with_w:
        out_type.append(jax.ShapeDtypeStruct((B * F,), jnp.float32))
        scratch += [pltpu.VMEM((rows_per_w,), jnp.float32),
                    pltpu.SemaphoreType.DMA]

    @functools.partial(
        pl.kernel,
        mesh=mesh,
        out_type=tuple(out_type),
        scratch_types=scratch,
        compiler_params=pltpu.CompilerParams(use_tc_tiling_on_sc=False,
                                            needs_layout_passes=False),
    )
    def gather_kernel(ids_hbm, *rest):
        tabs = rest[:DH]
        if with_w:
            (fmw_hbm, emb_out, wg_out,
             idx_v, gbuf, rows_v, sem_v, w_v, sem_w) = rest[DH:]
        else:
            (emb_out, idx_v, gbuf, rows_v, sem_v) = rest[DH:]
        wid = lax.axis_index("s") * nc + lax.axis_index("c")
        base = wid * rows_per_w
        pltpu.sync_copy(ids_hbm.at[pl.ds(base, rows_per_w)], idx_v)

        def issue(c):
            idx_c = idx_v.at[pl.ds(c * IDX_CHUNK, IDX_CHUNK)]
            if with_w:
                pltpu.async_copy(fmw_hbm.at[idx_c],
                                 w_v.at[pl.ds(c * IDX_CHUNK, IDX_CHUNK)],
                                 sem_w)
            for d in range(DH):
                pltpu.async_copy(
                    tabs[d].at[idx_c],
                    gbuf.at[pl.ds(d * rows_per_w + c * IDX_CHUNK, IDX_CHUNK)],
                    sem_v)

        def drain(c):
            if with_w:
                pltpu.make_async_copy(
                    fmw_hbm.at[pl.ds(0, IDX_CHUNK)],
                    w_v.at[pl.ds(c * IDX_CHUNK, IDX_CHUNK)], sem_w).wait()
            for d in range(DH):
                pltpu.make_async_copy(
                    tabs[d].at[pl.ds(0, IDX_CHUNK)],
                    gbuf.at[pl.ds(d * rows_per_w + c * IDX_CHUNK, IDX_CHUNK)],
                    sem_v).wait()

        def chunk_body(c, carry):
            issue(c)

            @pl.when(c > 1)
            def _():
                drain(c - 2)
            return carry

        lax.fori_loop(0, chunks, chunk_body, 0, unroll=False)
        drain(chunks - 2)
        drain(chunks - 1)

        d_iota = lax.iota(jnp.int32, 16)
        dmod = d_iota % DH
        jsel = (d_iota >= DH).astype(jnp.int32)
        ld_base = dmod * rows_per_w

        def tr_body(j, jvec):
            j2 = jvec + jsel
            row = plsc.load_gather(gbuf, [ld_base + j2])
            plsc.store_scatter(rows_v, [j2 * DH + dmod], row)
            return jvec + 2

        lax.fori_loop(0, rows_per_w // 2, tr_body,
                      jnp.zeros((16,), jnp.int32), unroll=4)

        pltpu.sync_copy(rows_v, emb_out.at[pl.ds(base * DH, rows_per_w * DH)])
        if with_w:
            pltpu.sync_copy(w_v, wg_out.at[pl.ds(base, rows_per_w)])

    return gather_kernel


BLK = 512


def _tc_body(lo_ref, hi_ref, vrep_ref, vals_ref, wg_ref, a_ref,
             w1_ref, b1_ref, w2_ref, b2_ref, w3_ref, scal_ref, out_ref):
    emb = jnp.concatenate([lo_ref[...], hi_ref[...]], axis=1) * vrep_ref[...]
    s = jnp.dot(emb, a_ref[...], preferred_element_type=jnp.float32)
    y_wxx = 0.5 * (jnp.sum(s * s, axis=1, keepdims=True)
                   - jnp.sum(emb * emb, axis=1, keepdims=True))
    y_wx = jnp.sum(vals_ref[...] * wg_ref[...], axis=1, keepdims=True)
    h = jnp.dot(emb, w1_ref[...], preferred_element_type=jnp.float32)
    h = jnp.maximum(h + b1_ref[...], 0.0)
    h = jnp.dot(h, w2_ref[...], preferred_element_type=jnp.float32)
    h = jnp.maximum(h + b2_ref[...], 0.0)
    y_d = jnp.sum(h * w3_ref[...], axis=1, keepdims=True)
    y = y_wx + y_wxx + y_d + scal_ref[...]
    out_ref[...] = 1.0 / (1.0 + jnp.exp(-y))


_PERM = np.array([f * D + d for f in range(F) for d in range(DH)]
                 + [f * D + d for f in range(F) for d in range(DH, D)],
                 dtype=np.int32)


def kernel(feat_ids, feat_vals, fm_b, fm_w, fm_v, W1, b1, W2, b2, W3, b3):
    ids_flat = feat_ids.astype(jnp.int32).reshape(B * F)
    fmvt = fm_v.T

    tabs_lo = _tc_detile(fmvt, 0)
    emb_lo, wg_flat = _sc_gather_kernel(True)(ids_flat, *tabs_lo, fm_w)
    tabs_hi = _tc_detile(fmvt, 1)
    (emb_hi,) = _sc_gather_kernel(False)(ids_flat, *tabs_hi)

    lo = emb_lo.reshape(B, FDH)
    hi = emb_hi.reshape(B, FDH)
    wg = wg_flat.reshape(B, F)

    vrep = jnp.tile(jnp.repeat(feat_vals, DH, axis=1), (1, 2))
    a_mat = jnp.tile(jnp.eye(D, dtype=jnp.float32), (F, 1))[_PERM]
    w1p = jnp.pad(W1, ((0, 0), (0, HP - H1)))[_PERM]
    b1p = jnp.pad(b1, (0, HP - H1)).reshape(1, HP)
    w2p = jnp.pad(W2, ((0, HP - H1), (0, HP - H2)))
    b2p = jnp.pad(b2, (0, HP - H2)).reshape(1, HP)
    w3p = jnp.pad(W3[:, 0], (0, HP - H2)).reshape(1, HP)
    scal = (fm_b + b3).reshape(1, 1)

    full = lambda shape: pl.BlockSpec(shape, lambda i: (0, 0))
    preds = pl.pallas_call(
        _tc_body,
        grid=(B // BLK,),
        in_specs=[
            pl.BlockSpec((BLK, FDH), lambda i: (i, 0)),
            pl.BlockSpec((BLK, FDH), lambda i: (i, 0)),
            pl.BlockSpec((BLK, FD), lambda i: (i, 0)),
            pl.BlockSpec((BLK, F), lambda i: (i, 0)),
            pl.BlockSpec((BLK, F), lambda i: (i, 0)),
            full((FD, D)),
            full((FD, HP)),
            full((1, HP)),
            full((HP, HP)),
            full((1, HP)),
            full((1, HP)),
            full((1, 1)),
        ],
        out_specs=pl.BlockSpec((BLK, 1), lambda i: (i, 0)),
        out_shape=jax.ShapeDtypeStruct((B, 1), jnp.float32),
        compiler_params=pltpu.CompilerParams(
            dimension_semantics=("parallel",)),
    )(lo, hi, vrep, feat_vals, wg, a_mat, w1p, b1p, w2p, b2p, w3p, scal)
    return preds.reshape(-1)
